# baseline (speedup 1.0000x reference)
import functools

import jax
import jax.numpy as jnp
from jax import lax
from jax.experimental import pallas as pl
from jax.experimental.pallas import tpu as pltpu
from jax.experimental.pallas import tpu_sc as plsc

_BIG = 1e10
_INF = float("inf")
_PAD = 16
_CH = 32
_NSEL = 32


def _fps_body(xyz_ref, cent_ref, *, n_centers):
    x = xyz_ref[0]
    y = xyz_ref[1]
    z = xyz_ref[2]
    B, N = x.shape
    iota = jax.lax.broadcasted_iota(jnp.int32, (B, N), 1)
    iota_g = jax.lax.broadcasted_iota(jnp.int32, (B, n_centers), 1)

    def body(i, state):
        dists, lx, ly, lz, ax, ay, az = state
        dx = x - lx
        dy = y - ly
        dz = z - lz
        d = dx * dx + dy * dy
        d = d + dz * dz
        dists = jnp.minimum(dists, d)
        m = jnp.max(dists, axis=1, keepdims=True)
        j = jnp.min(jnp.where(dists == m, iota, N), axis=1, keepdims=True)
        onehot = iota == j
        lx = jnp.sum(jnp.where(onehot, x, 0.0), axis=1, keepdims=True)
        ly = jnp.sum(jnp.where(onehot, y, 0.0), axis=1, keepdims=True)
        lz = jnp.sum(jnp.where(onehot, z, 0.0), axis=1, keepdims=True)
        sel = iota_g == i
        ax = ax + jnp.where(sel, lx, 0.0)
        ay = ay + jnp.where(sel, ly, 0.0)
        az = az + jnp.where(sel, lz, 0.0)
        return (dists, lx, ly, lz, ax, ay, az)

    dists0 = jnp.full((B, N), _BIG, dtype=jnp.float32)
    sel0 = iota_g == 0
    ax0 = jnp.where(sel0, x[:, 0:1], 0.0)
    ay0 = jnp.where(sel0, y[:, 0:1], 0.0)
    az0 = jnp.where(sel0, z[:, 0:1], 0.0)
    out = jax.lax.fori_loop(
        1, n_centers, body,
        (dists0, x[:, 0:1], y[:, 0:1], z[:, 0:1], ax0, ay0, az0))
    cent_ref[0] = out[4]
    cent_ref[1] = out[5]
    cent_ref[2] = out[6]


def _chunksel_body(xyz_ref, cent_ref, chunk_ref, *, n_chunks):
    x = xyz_ref[0][:, 0:1]
    y = xyz_ref[0][:, 1:2]
    z = xyz_ref[0][:, 2:3]
    cx = cent_ref[0, 0:1, :]
    cy = cent_ref[0, 1:2, :]
    cz = cent_ref[0, 2:3, :]
    N = x.shape[0]
    G = cx.shape[1]
    b = pl.program_id(0)

    dx = x - cx
    dy = y - cy
    dz = z - cz
    d = dx * dx + dy * dy
    d = d + dz * dz

    m = jnp.min(d.reshape(n_chunks, _CH, G), axis=1)
    iota_c = jax.lax.broadcasted_iota(jnp.int32, (n_chunks, G), 0)
    iota_slot = jax.lax.broadcasted_iota(jnp.int32, (_NSEL, G), 0)

    acc = jnp.zeros((_NSEL, G), dtype=jnp.int32)
    for s in range(_NSEL):
        mv = jnp.min(m, axis=0, keepdims=True)
        j = jnp.min(jnp.where(m == mv, iota_c, n_chunks), axis=0,
                    keepdims=True)
        acc = acc + jnp.where(iota_slot == s, j, 0)
        if s + 1 < _NSEL:
            m = jnp.where(iota_c == j, _INF, m)
    chunk_ref[0] = acc


def _extract_body(candx_ref, candy_ref, candz_ref, ic_ref, cent_ref,
                  idx_ref, *, group_size, row_block, n_points):
    b = pl.program_id(0)
    candx = candx_ref[0]
    candy = candy_ref[0]
    candz = candz_ref[0]
    ic = ic_ref[0]
    cx = cent_ref[0][:, 0:1]
    cy = cent_ref[0][:, 1:2]
    cz = cent_ref[0][:, 2:3]

    dx = candx - cx
    dy = candy - cy
    dz = candz - cz
    dc = dx * dx + dy * dy
    dc = dc + dz * dz

    iota_s = jax.lax.broadcasted_iota(jnp.int32, (row_block, group_size), 1)
    acc = jnp.zeros((row_block, group_size), dtype=jnp.int32)
    for s in range(group_size):
        mv = jnp.min(dc, axis=1, keepdims=True)
        j = jnp.min(jnp.where(dc == mv, ic, n_points), axis=1, keepdims=True)
        acc = acc + jnp.where(iota_s == s, j, 0)
        if s + 1 < group_size:
            dc = jnp.where(ic == j, _INF, dc)
    idx_ref[0] = acc + b * n_points


def _sub_body(rows_ref, cent_ref, out_ref):
    out_ref[0] = rows_ref[0] - cent_ref[0][:, None, :]


def _sc_gather(table, idx2d, row_w, buf_rows=128):
    info = plsc.get_sparse_core_info()
    nc, ns = info.num_cores, info.num_subcores
    nw = nc * ns
    total = idx2d.shape[0] * 128
    rows_per_w = total // nw
    chunks_per_w = rows_per_w // 128
    mesh = plsc.VectorSubcoreMesh(core_axis_name="c", subcore_axis_name="s")

    @functools.partial(
        pl.kernel, mesh=mesh,
        compiler_params=pltpu.CompilerParams(use_tc_tiling_on_sc=False),
        out_type=jax.ShapeDtypeStruct((total, row_w), jnp.float32),
        scratch_types=[
            pltpu.VMEM((chunks_per_w, 128), jnp.int32),
            pltpu.VMEM((2, buf_rows, row_w), jnp.float32),
            pltpu.SemaphoreType.DMA,
            pltpu.SemaphoreType.DMA,
        ],
    )
    def k(table_hbm, idx_hbm, out_hbm, idx_v, rows_v, sem0, sem1):
        wid = lax.axis_index("s") * nc + lax.axis_index("c")
        pltpu.sync_copy(idx_hbm.at[pl.ds(wid * chunks_per_w, chunks_per_w)],
                        idx_v)
        sems = [sem0, sem1]
        gathers = []
        outs = []
        for c in range(chunks_per_w):
            p = c % 2
            if c >= 2:
                outs[c - 2].wait()
            gathers.append(pltpu.async_copy(
                table_hbm.at[idx_v.at[c]], rows_v.at[p], sems[p]))
            gathers[c].wait()
            outs.append(pltpu.async_copy(
                rows_v.at[p],
                out_hbm.at[pl.ds(wid * rows_per_w + c * 128, 128)],
                sems[p]))
        outs[chunks_per_w - 2].wait()
        outs[chunks_per_w - 1].wait()

    return k(table, idx2d)


def _group(xyz, n_centers, group_size, interpret=False):
    B, N, _ = xyz.shape
    n_chunks = N // _CH
    xyzT = jnp.transpose(xyz, (2, 0, 1))

    FB = 2
    cent3 = pl.pallas_call(
        functools.partial(_fps_body, n_centers=n_centers),
        grid=(FB,),
        in_specs=[pl.BlockSpec((3, B // FB, N), lambda i: (0, i, 0))],
        out_specs=pl.BlockSpec((3, B // FB, n_centers), lambda i: (0, i, 0)),
        out_shape=jax.ShapeDtypeStruct((3, B, n_centers), jnp.float32),
        compiler_params=pltpu.CompilerParams(
            dimension_semantics=("parallel",),
        ),
        interpret=interpret,
    )(xyzT)

    centb = jnp.transpose(cent3, (1, 0, 2))
    center = jnp.transpose(cent3, (1, 2, 0))

    chunks = pl.pallas_call(
        functools.partial(_chunksel_body, n_chunks=n_chunks),
        grid=(B,),
        in_specs=[
            pl.BlockSpec((1, N, 3), lambda b: (b, 0, 0)),
            pl.BlockSpec((1, 3, n_centers), lambda b: (b, 0, 0)),
        ],
        out_specs=pl.BlockSpec((1, _NSEL, n_centers), lambda b: (b, 0, 0)),
        out_shape=jax.ShapeDtypeStruct((B, _NSEL, n_centers), jnp.int32),
        compiler_params=pltpu.CompilerParams(
            dimension_semantics=("parallel",),
        ),
        interpret=interpret,
    )(xyz, centb)

    chunks_l = jnp.transpose(chunks, (0, 2, 1))

    ncand = _NSEL * _CH
    table3 = xyzT.reshape(3 * B * n_chunks, _CH)
    cg = chunks_l + (jnp.arange(B, dtype=jnp.int32) * n_chunks)[:, None,
                                                                None]
    idx3 = jnp.stack([cg, cg + B * n_chunks, cg + 2 * B * n_chunks])
    cand = _sc_gather(table3, idx3.reshape(-1, 128), _CH)
    cand4 = cand.reshape(3, B, n_centers, ncand)

    ic_arr = (chunks_l[..., None] * _CH
              + jnp.arange(_CH, dtype=jnp.int32)).reshape(
                  B, n_centers, ncand)

    RB = min(256, n_centers)
    idx = pl.pallas_call(
        functools.partial(_extract_body, group_size=group_size,
                          row_block=RB, n_points=N),
        grid=(B, n_centers // RB),
        in_specs=[
            pl.BlockSpec((1, RB, ncand), lambda b, r: (b, r, 0)),
            pl.BlockSpec((1, RB, ncand), lambda b, r: (b, r, 0)),
            pl.BlockSpec((1, RB, ncand), lambda b, r: (b, r, 0)),
            pl.BlockSpec((1, RB, ncand), lambda b, r: (b, r, 0)),
            pl.BlockSpec((1, RB, 3), lambda b, r: (b, r, 0)),
        ],
        out_specs=pl.BlockSpec((1, RB, group_size), lambda b, r: (b, r, 0)),
        out_shape=jax.ShapeDtypeStruct((B, n_centers, group_size),
                                       jnp.int32),
        compiler_params=pltpu.CompilerParams(
            dimension_semantics=("parallel", "parallel"),
        ),
        interpret=interpret,
    )(cand4[0], cand4[1], cand4[2], ic_arr, center)

    table = jnp.pad(xyz.reshape(B * N, 3), ((0, 0), (0, _PAD - 3)))
    rows = _sc_gather(table, idx.reshape(-1, 128), _PAD)
    rows4 = rows.reshape(B, n_centers, group_size, _PAD)

    cent_p = jnp.pad(center, ((0, 0), (0, 0), (0, _PAD - 3)))

    nb = pl.pallas_call(
        _sub_body,
        grid=(B,),
        in_specs=[
            pl.BlockSpec((1, n_centers, group_size, _PAD),
                         lambda b: (b, 0, 0, 0)),
            pl.BlockSpec((1, n_centers, _PAD), lambda b: (b, 0, 0)),
        ],
        out_specs=pl.BlockSpec((1, n_centers, group_size, _PAD),
                               lambda b: (b, 0, 0, 0)),
        out_shape=jax.ShapeDtypeStruct((B, n_centers, group_size, _PAD),
                                       jnp.float32),
        compiler_params=pltpu.CompilerParams(
            dimension_semantics=("parallel",),
        ),
        interpret=interpret,
    )(rows4, cent_p)

    return (nb[..., :3], center)


def kernel(xyz):
    return _group(xyz, 256, 32)


# speedup vs baseline: 1.1052x; 1.1052x over previous
import functools

import jax
import jax.numpy as jnp
from jax import lax
from jax.experimental import pallas as pl
from jax.experimental.pallas import tpu as pltpu
from jax.experimental.pallas import tpu_sc as plsc

_BIG = 1e10
_INF = float("inf")
_PAD = 16
_CH = 32
_NSEL = 32


def _fps_body(xyz_ref, cent_ref, *, n_centers):
    x = xyz_ref[0]
    y = xyz_ref[1]
    z = xyz_ref[2]
    B, N = x.shape
    iota = jax.lax.broadcasted_iota(jnp.int32, (B, N), 1)
    iota_g = jax.lax.broadcasted_iota(jnp.int32, (B, n_centers), 1)

    def body(i, state):
        dists, lx, ly, lz, ax, ay, az = state
        dx = x - lx
        dy = y - ly
        dz = z - lz
        d = dx * dx + dy * dy
        d = d + dz * dz
        dists = jnp.minimum(dists, d)
        m = jnp.max(dists, axis=1, keepdims=True)
        j = jnp.min(jnp.where(dists == m, iota, N), axis=1, keepdims=True)
        onehot = iota == j
        lx = jnp.sum(jnp.where(onehot, x, 0.0), axis=1, keepdims=True)
        ly = jnp.sum(jnp.where(onehot, y, 0.0), axis=1, keepdims=True)
        lz = jnp.sum(jnp.where(onehot, z, 0.0), axis=1, keepdims=True)
        sel = iota_g == i
        ax = ax + jnp.where(sel, lx, 0.0)
        ay = ay + jnp.where(sel, ly, 0.0)
        az = az + jnp.where(sel, lz, 0.0)
        return (dists, lx, ly, lz, ax, ay, az)

    dists0 = jnp.full((B, N), _BIG, dtype=jnp.float32)
    sel0 = iota_g == 0
    ax0 = jnp.where(sel0, x[:, 0:1], 0.0)
    ay0 = jnp.where(sel0, y[:, 0:1], 0.0)
    az0 = jnp.where(sel0, z[:, 0:1], 0.0)
    out = jax.lax.fori_loop(
        1, n_centers, body,
        (dists0, x[:, 0:1], y[:, 0:1], z[:, 0:1], ax0, ay0, az0))
    cent_ref[0] = out[4]
    cent_ref[1] = out[5]
    cent_ref[2] = out[6]


def _chunksel_body(xyz_ref, cent_ref, chunk_ref, *, n_chunks):
    x = xyz_ref[0][:, 0:1]
    y = xyz_ref[0][:, 1:2]
    z = xyz_ref[0][:, 2:3]
    cx = cent_ref[0, 0:1, :]
    cy = cent_ref[0, 1:2, :]
    cz = cent_ref[0, 2:3, :]
    N = x.shape[0]
    G = cx.shape[1]
    b = pl.program_id(0)

    dx = x - cx
    dy = y - cy
    dz = z - cz
    d = dx * dx + dy * dy
    d = d + dz * dz

    m = jnp.min(d.reshape(n_chunks, _CH, G), axis=1)
    iota_c = jax.lax.broadcasted_iota(jnp.int32, (n_chunks, G), 0)
    iota_slot = jax.lax.broadcasted_iota(jnp.int32, (_NSEL, G), 0)

    acc = jnp.zeros((_NSEL, G), dtype=jnp.int32)
    for s in range(_NSEL):
        mv = jnp.min(m, axis=0, keepdims=True)
        j = jnp.min(jnp.where(m == mv, iota_c, n_chunks), axis=0,
                    keepdims=True)
        acc = acc + jnp.where(iota_slot == s, j, 0)
        if s + 1 < _NSEL:
            m = jnp.where(iota_c == j, _INF, m)
    chunk_ref[0] = acc


def _extract_body(candx_ref, candy_ref, candz_ref, ic_ref, cent_ref,
                  idx_ref, *, group_size, row_block, n_points):
    b = pl.program_id(0)
    candx = candx_ref[0]
    candy = candy_ref[0]
    candz = candz_ref[0]
    ic = ic_ref[0]
    cx = cent_ref[0][:, 0:1]
    cy = cent_ref[0][:, 1:2]
    cz = cent_ref[0][:, 2:3]

    dx = candx - cx
    dy = candy - cy
    dz = candz - cz
    dc = dx * dx + dy * dy
    dc = dc + dz * dz

    iota_s = jax.lax.broadcasted_iota(jnp.int32, (row_block, group_size), 1)
    acc = jnp.zeros((row_block, group_size), dtype=jnp.int32)
    for s in range(group_size):
        mv = jnp.min(dc, axis=1, keepdims=True)
        j = jnp.min(jnp.where(dc == mv, ic, n_points), axis=1, keepdims=True)
        acc = acc + jnp.where(iota_s == s, j, 0)
        if s + 1 < group_size:
            dc = jnp.where(ic == j, _INF, dc)
    idx_ref[0] = acc + b * n_points


def _sub_body(rows_ref, cent_ref, out_ref):
    out_ref[0] = rows_ref[0] - cent_ref[0][:, None, :]


def _sc_gather(table, idx2d, row_w, buf_rows=128):
    info = plsc.get_sparse_core_info()
    nc, ns = info.num_cores, info.num_subcores
    nw = nc * ns
    total = idx2d.shape[0] * 128
    rows_per_w = total // nw
    chunks_per_w = rows_per_w // 128
    mesh = plsc.VectorSubcoreMesh(core_axis_name="c", subcore_axis_name="s")

    @functools.partial(
        pl.kernel, mesh=mesh,
        compiler_params=pltpu.CompilerParams(use_tc_tiling_on_sc=False),
        out_type=jax.ShapeDtypeStruct((total, row_w), jnp.float32),
        scratch_types=[
            pltpu.VMEM((chunks_per_w, 128), jnp.int32),
            pltpu.VMEM((2, buf_rows, row_w), jnp.float32),
            pltpu.SemaphoreType.DMA,
            pltpu.SemaphoreType.DMA,
        ],
    )
    def k(table_hbm, idx_hbm, out_hbm, idx_v, rows_v, sem0, sem1):
        wid = lax.axis_index("s") * nc + lax.axis_index("c")
        pltpu.sync_copy(idx_hbm.at[pl.ds(wid * chunks_per_w, chunks_per_w)],
                        idx_v)
        sems = [sem0, sem1]
        gathers = []
        outs = []
        for c in range(chunks_per_w):
            p = c % 2
            if c >= 2:
                outs[c - 2].wait()
            gathers.append(pltpu.async_copy(
                table_hbm.at[idx_v.at[c]], rows_v.at[p], sems[p]))
            gathers[c].wait()
            outs.append(pltpu.async_copy(
                rows_v.at[p],
                out_hbm.at[pl.ds(wid * rows_per_w + c * 128, 128)],
                sems[p]))
        outs[chunks_per_w - 2].wait()
        outs[chunks_per_w - 1].wait()

    return k(table, idx2d)


def _group(xyz, n_centers, group_size, interpret=False):
    B, N, _ = xyz.shape
    n_chunks = N // _CH
    xyzT = jnp.transpose(xyz, (2, 0, 1))

    cent3 = pl.pallas_call(
        functools.partial(_fps_body, n_centers=n_centers),
        out_shape=jax.ShapeDtypeStruct((3, B, n_centers), jnp.float32),
        interpret=interpret,
    )(xyzT)

    centb = jnp.transpose(cent3, (1, 0, 2))
    center = jnp.transpose(cent3, (1, 2, 0))

    chunks = pl.pallas_call(
        functools.partial(_chunksel_body, n_chunks=n_chunks),
        grid=(B,),
        in_specs=[
            pl.BlockSpec((1, N, 3), lambda b: (b, 0, 0)),
            pl.BlockSpec((1, 3, n_centers), lambda b: (b, 0, 0)),
        ],
        out_specs=pl.BlockSpec((1, _NSEL, n_centers), lambda b: (b, 0, 0)),
        out_shape=jax.ShapeDtypeStruct((B, _NSEL, n_centers), jnp.int32),
        compiler_params=pltpu.CompilerParams(
            dimension_semantics=("parallel",),
        ),
        interpret=interpret,
    )(xyz, centb)

    chunks_l = jnp.transpose(chunks, (0, 2, 1))

    ncand = _NSEL * _CH
    table3 = xyzT.reshape(3 * B * n_chunks, _CH)
    cg = chunks_l + (jnp.arange(B, dtype=jnp.int32) * n_chunks)[:, None,
                                                                None]
    idx3 = jnp.stack([cg, cg + B * n_chunks, cg + 2 * B * n_chunks])
    cand = _sc_gather(table3, idx3.reshape(-1, 128), _CH)
    cand4 = cand.reshape(3, B, n_centers, ncand)

    ic_arr = (chunks_l[..., None] * _CH
              + jnp.arange(_CH, dtype=jnp.int32)).reshape(
                  B, n_centers, ncand)

    RB = min(256, n_centers)
    idx = pl.pallas_call(
        functools.partial(_extract_body, group_size=group_size,
                          row_block=RB, n_points=N),
        grid=(B, n_centers // RB),
        in_specs=[
            pl.BlockSpec((1, RB, ncand), lambda b, r: (b, r, 0)),
            pl.BlockSpec((1, RB, ncand), lambda b, r: (b, r, 0)),
            pl.BlockSpec((1, RB, ncand), lambda b, r: (b, r, 0)),
            pl.BlockSpec((1, RB, ncand), lambda b, r: (b, r, 0)),
            pl.BlockSpec((1, RB, 3), lambda b, r: (b, r, 0)),
        ],
        out_specs=pl.BlockSpec((1, RB, group_size), lambda b, r: (b, r, 0)),
        out_shape=jax.ShapeDtypeStruct((B, n_centers, group_size),
                                       jnp.int32),
        compiler_params=pltpu.CompilerParams(
            dimension_semantics=("parallel", "parallel"),
        ),
        interpret=interpret,
    )(cand4[0], cand4[1], cand4[2], ic_arr, center)

    rows = _sc_gather(xyz.reshape(B * N, 3), idx.reshape(-1, 128), 3)
    rows4 = rows.reshape(B, n_centers, group_size, 3)

    nb = pl.pallas_call(
        _sub_body,
        grid=(B,),
        in_specs=[
            pl.BlockSpec((1, n_centers, group_size, 3),
                         lambda b: (b, 0, 0, 0)),
            pl.BlockSpec((1, n_centers, 3), lambda b: (b, 0, 0)),
        ],
        out_specs=pl.BlockSpec((1, n_centers, group_size, 3),
                               lambda b: (b, 0, 0, 0)),
        out_shape=jax.ShapeDtypeStruct((B, n_centers, group_size, 3),
                                       jnp.float32),
        compiler_params=pltpu.CompilerParams(
            dimension_semantics=("parallel",),
        ),
        interpret=interpret,
    )(rows4, center)

    return (nb, center)


def kernel(xyz):
    return _group(xyz, 256, 32)


# speedup vs baseline: 1.1054x; 1.0002x over previous
import functools

import jax
import jax.numpy as jnp
from jax import lax
from jax.experimental import pallas as pl
from jax.experimental.pallas import tpu as pltpu
from jax.experimental.pallas import tpu_sc as plsc

_BIG = 1e10
_INF = float("inf")
_PAD = 16
_CH = 32
_NSEL = 32


def _fps_body(xyz_ref, cent_ref, *, n_centers):
    x = xyz_ref[0]
    y = xyz_ref[1]
    z = xyz_ref[2]
    B, N = x.shape
    iota = jax.lax.broadcasted_iota(jnp.int32, (B, N), 1)
    iota_g = jax.lax.broadcasted_iota(jnp.int32, (B, n_centers), 1)

    def body(i, state):
        dists, lx, ly, lz, ax, ay, az = state
        dx = x - lx
        dy = y - ly
        dz = z - lz
        d = dx * dx + dy * dy
        d = d + dz * dz
        dists = jnp.minimum(dists, d)
        m = jnp.max(dists, axis=1, keepdims=True)
        j = jnp.min(jnp.where(dists == m, iota, N), axis=1, keepdims=True)
        onehot = iota == j
        lx = jnp.sum(jnp.where(onehot, x, 0.0), axis=1, keepdims=True)
        ly = jnp.sum(jnp.where(onehot, y, 0.0), axis=1, keepdims=True)
        lz = jnp.sum(jnp.where(onehot, z, 0.0), axis=1, keepdims=True)
        sel = iota_g == i
        ax = ax + jnp.where(sel, lx, 0.0)
        ay = ay + jnp.where(sel, ly, 0.0)
        az = az + jnp.where(sel, lz, 0.0)
        return (dists, lx, ly, lz, ax, ay, az)

    dists0 = jnp.full((B, N), _BIG, dtype=jnp.float32)
    sel0 = iota_g == 0
    ax0 = jnp.where(sel0, x[:, 0:1], 0.0)
    ay0 = jnp.where(sel0, y[:, 0:1], 0.0)
    az0 = jnp.where(sel0, z[:, 0:1], 0.0)
    out = jax.lax.fori_loop(
        1, n_centers, body,
        (dists0, x[:, 0:1], y[:, 0:1], z[:, 0:1], ax0, ay0, az0))
    cent_ref[0] = out[4]
    cent_ref[1] = out[5]
    cent_ref[2] = out[6]


def _chunksel_body(xyz_ref, cent_ref, chunk_ref, *, n_chunks):
    x = xyz_ref[0][:, 0:1]
    y = xyz_ref[0][:, 1:2]
    z = xyz_ref[0][:, 2:3]
    cx = cent_ref[0, 0:1, :]
    cy = cent_ref[0, 1:2, :]
    cz = cent_ref[0, 2:3, :]
    N = x.shape[0]
    G = cx.shape[1]
    b = pl.program_id(0)

    dx = x - cx
    dy = y - cy
    dz = z - cz
    d = dx * dx + dy * dy
    d = d + dz * dz

    m = jnp.min(d.reshape(n_chunks, _CH, G), axis=1)
    iota_c = jax.lax.broadcasted_iota(jnp.int32, (n_chunks, G), 0)
    iota_slot = jax.lax.broadcasted_iota(jnp.int32, (_NSEL, G), 0)

    acc = jnp.zeros((_NSEL, G), dtype=jnp.int32)
    for s in range(_NSEL):
        mv = jnp.min(m, axis=0, keepdims=True)
        j = jnp.min(jnp.where(m == mv, iota_c, n_chunks), axis=0,
                    keepdims=True)
        acc = acc + jnp.where(iota_slot == s, j, 0)
        if s + 1 < _NSEL:
            m = jnp.where(iota_c == j, _INF, m)
    chunk_ref[0] = acc


def _extract_body(candx_ref, candy_ref, candz_ref, ic_ref, cent_ref,
                  idx_ref, *, group_size, row_block, n_points):
    b = pl.program_id(0)
    candx = candx_ref[0]
    candy = candy_ref[0]
    candz = candz_ref[0]
    ic = ic_ref[0]
    cx = cent_ref[0][:, 0:1]
    cy = cent_ref[0][:, 1:2]
    cz = cent_ref[0][:, 2:3]

    dx = candx - cx
    dy = candy - cy
    dz = candz - cz
    dc = dx * dx + dy * dy
    dc = dc + dz * dz

    iota_s = jax.lax.broadcasted_iota(jnp.int32, (row_block, group_size), 1)
    acc = jnp.zeros((row_block, group_size), dtype=jnp.int32)
    for s in range(group_size):
        mv = jnp.min(dc, axis=1, keepdims=True)
        j = jnp.min(jnp.where(dc == mv, ic, n_points), axis=1, keepdims=True)
        acc = acc + jnp.where(iota_s == s, j, 0)
        if s + 1 < group_size:
            dc = jnp.where(ic == j, _INF, dc)
    idx_ref[0] = acc + b * n_points


def _sub_body(rows_ref, cent_ref, out_ref):
    out_ref[0] = rows_ref[0] - cent_ref[0][:, None, :]


def _sc_gather(table, idx2d, row_w, buf_rows=128):
    info = plsc.get_sparse_core_info()
    nc, ns = info.num_cores, info.num_subcores
    nw = nc * ns
    total = idx2d.shape[0] * 128
    rows_per_w = total // nw
    chunks_per_w = rows_per_w // 128
    mesh = plsc.VectorSubcoreMesh(core_axis_name="c", subcore_axis_name="s")

    @functools.partial(
        pl.kernel, mesh=mesh,
        compiler_params=pltpu.CompilerParams(use_tc_tiling_on_sc=False),
        out_type=jax.ShapeDtypeStruct((total, row_w), jnp.float32),
        scratch_types=[
            pltpu.VMEM((chunks_per_w, 128), jnp.int32),
            pltpu.VMEM((2, buf_rows, row_w), jnp.float32),
            pltpu.SemaphoreType.DMA,
            pltpu.SemaphoreType.DMA,
        ],
    )
    def k(table_hbm, idx_hbm, out_hbm, idx_v, rows_v, sem0, sem1):
        wid = lax.axis_index("s") * nc + lax.axis_index("c")
        pltpu.sync_copy(idx_hbm.at[pl.ds(wid * chunks_per_w, chunks_per_w)],
                        idx_v)
        sems = [sem0, sem1]
        gathers = []
        outs = []
        for c in range(chunks_per_w):
            p = c % 2
            if c >= 2:
                outs[c - 2].wait()
            gathers.append(pltpu.async_copy(
                table_hbm.at[idx_v.at[c]], rows_v.at[p], sems[p]))
            gathers[c].wait()
            outs.append(pltpu.async_copy(
                rows_v.at[p],
                out_hbm.at[pl.ds(wid * rows_per_w + c * 128, 128)],
                sems[p]))
        outs[chunks_per_w - 2].wait()
        outs[chunks_per_w - 1].wait()

    return k(table, idx2d)


def _group(xyz, n_centers, group_size, interpret=False):
    B, N, _ = xyz.shape
    n_chunks = N // _CH
    xyzT = jnp.transpose(xyz, (2, 0, 1))

    cent3 = pl.pallas_call(
        functools.partial(_fps_body, n_centers=n_centers),
        out_shape=jax.ShapeDtypeStruct((3, B, n_centers), jnp.float32),
        interpret=interpret,
    )(xyzT)

    centb = jnp.transpose(cent3, (1, 0, 2))
    center = jnp.transpose(cent3, (1, 2, 0))

    chunks = pl.pallas_call(
        functools.partial(_chunksel_body, n_chunks=n_chunks),
        grid=(B,),
        in_specs=[
            pl.BlockSpec((1, N, 3), lambda b: (b, 0, 0)),
            pl.BlockSpec((1, 3, n_centers), lambda b: (b, 0, 0)),
        ],
        out_specs=pl.BlockSpec((1, _NSEL, n_centers), lambda b: (b, 0, 0)),
        out_shape=jax.ShapeDtypeStruct((B, _NSEL, n_centers), jnp.int32),
        compiler_params=pltpu.CompilerParams(
            dimension_semantics=("parallel",),
        ),
        interpret=interpret,
    )(xyz, centb)

    chunks_l = jnp.transpose(chunks, (0, 2, 1))

    ncand = _NSEL * _CH
    table3 = xyzT.reshape(3 * B * n_chunks, _CH)
    cg = chunks_l + (jnp.arange(B, dtype=jnp.int32) * n_chunks)[:, None,
                                                                None]
    idx3 = jnp.stack([cg, cg + B * n_chunks, cg + 2 * B * n_chunks])
    cand = _sc_gather(table3, idx3.reshape(-1, 128), _CH)
    cand4 = cand.reshape(3, B, n_centers, ncand)

    ic_arr = (chunks_l[..., None] * _CH
              + jnp.arange(_CH, dtype=jnp.int32)).reshape(
                  B, n_centers, ncand)

    RB = min(256, n_centers)
    idx = pl.pallas_call(
        functools.partial(_extract_body, group_size=group_size,
                          row_block=RB, n_points=N),
        grid=(B, n_centers // RB),
        in_specs=[
            pl.BlockSpec((1, RB, ncand), lambda b, r: (b, r, 0)),
            pl.BlockSpec((1, RB, ncand), lambda b, r: (b, r, 0)),
            pl.BlockSpec((1, RB, ncand), lambda b, r: (b, r, 0)),
            pl.BlockSpec((1, RB, ncand), lambda b, r: (b, r, 0)),
            pl.BlockSpec((1, RB, 3), lambda b, r: (b, r, 0)),
        ],
        out_specs=pl.BlockSpec((1, RB, group_size), lambda b, r: (b, r, 0)),
        out_shape=jax.ShapeDtypeStruct((B, n_centers, group_size),
                                       jnp.int32),
        compiler_params=pltpu.CompilerParams(
            dimension_semantics=("parallel", "parallel"),
        ),
        interpret=interpret,
    )(cand4[0], cand4[1], cand4[2], ic_arr, center)

    table = jnp.pad(xyz.reshape(B * N, 3), ((0, 0), (0, _PAD - 3)))
    rows = _sc_gather(table, idx.reshape(-1, 128), _PAD)
    rows4 = rows.reshape(B, n_centers, group_size, _PAD)

    cent_p = jnp.pad(center, ((0, 0), (0, 0), (0, _PAD - 3)))

    nb = pl.pallas_call(
        _sub_body,
        grid=(B,),
        in_specs=[
            pl.BlockSpec((1, n_centers, group_size, _PAD),
                         lambda b: (b, 0, 0, 0)),
            pl.BlockSpec((1, n_centers, _PAD), lambda b: (b, 0, 0)),
        ],
        out_specs=pl.BlockSpec((1, n_centers, group_size, _PAD),
                               lambda b: (b, 0, 0, 0)),
        out_shape=jax.ShapeDtypeStruct((B, n_centers, group_size, _PAD),
                                       jnp.float32),
        compiler_params=pltpu.CompilerParams(
            dimension_semantics=("parallel",),
        ),
        interpret=interpret,
    )(rows4, cent_p)

    return (nb[..., :3], center)


def kernel(xyz):
    return _group(xyz, 256, 32)


# speedup vs baseline: 1.1260x; 1.0187x over previous
import functools

import jax
import jax.numpy as jnp
from jax import lax
from jax.experimental import pallas as pl
from jax.experimental.pallas import tpu as pltpu
from jax.experimental.pallas import tpu_sc as plsc

_BIG = 1e10
_INF = float("inf")
_PAD = 16
_CH = 32
_NSEL = 32


def _fps_body(xyz_ref, cent_ref, *, n_centers):
    x = xyz_ref[0]
    y = xyz_ref[1]
    z = xyz_ref[2]
    B, N = x.shape
    iota = jax.lax.broadcasted_iota(jnp.int32, (B, N), 1)
    iota_g = jax.lax.broadcasted_iota(jnp.int32, (B, n_centers), 1)

    def body(i, state):
        dists, lx, ly, lz, ax, ay, az = state
        dx = x - lx
        dy = y - ly
        dz = z - lz
        d = dx * dx + dy * dy
        d = d + dz * dz
        dists = jnp.minimum(dists, d)
        j = jnp.argmax(dists, axis=1).astype(jnp.int32)[:, None]
        onehot = iota == j
        lx = jnp.sum(jnp.where(onehot, x, 0.0), axis=1, keepdims=True)
        ly = jnp.sum(jnp.where(onehot, y, 0.0), axis=1, keepdims=True)
        lz = jnp.sum(jnp.where(onehot, z, 0.0), axis=1, keepdims=True)
        sel = iota_g == i
        ax = ax + jnp.where(sel, lx, 0.0)
        ay = ay + jnp.where(sel, ly, 0.0)
        az = az + jnp.where(sel, lz, 0.0)
        return (dists, lx, ly, lz, ax, ay, az)

    dists0 = jnp.full((B, N), _BIG, dtype=jnp.float32)
    sel0 = iota_g == 0
    ax0 = jnp.where(sel0, x[:, 0:1], 0.0)
    ay0 = jnp.where(sel0, y[:, 0:1], 0.0)
    az0 = jnp.where(sel0, z[:, 0:1], 0.0)
    out = jax.lax.fori_loop(
        1, n_centers, body,
        (dists0, x[:, 0:1], y[:, 0:1], z[:, 0:1], ax0, ay0, az0))
    cent_ref[0] = out[4]
    cent_ref[1] = out[5]
    cent_ref[2] = out[6]


def _chunksel_body(xyz_ref, cent_ref, chunk_ref, *, n_chunks):
    x = xyz_ref[0][:, 0:1]
    y = xyz_ref[0][:, 1:2]
    z = xyz_ref[0][:, 2:3]
    cx = cent_ref[0, 0:1, :]
    cy = cent_ref[0, 1:2, :]
    cz = cent_ref[0, 2:3, :]
    N = x.shape[0]
    G = cx.shape[1]
    b = pl.program_id(0)

    dx = x - cx
    dy = y - cy
    dz = z - cz
    d = dx * dx + dy * dy
    d = d + dz * dz

    m = jnp.min(d.reshape(n_chunks, _CH, G), axis=1)
    iota_c = jax.lax.broadcasted_iota(jnp.int32, (n_chunks, G), 0)
    iota_slot = jax.lax.broadcasted_iota(jnp.int32, (_NSEL, G), 0)

    acc = jnp.zeros((_NSEL, G), dtype=jnp.int32)
    for s in range(_NSEL):
        mv = jnp.min(m, axis=0, keepdims=True)
        j = jnp.min(jnp.where(m == mv, iota_c, n_chunks), axis=0,
                    keepdims=True)
        acc = acc + jnp.where(iota_slot == s, j, 0)
        if s + 1 < _NSEL:
            m = jnp.where(iota_c == j, _INF, m)
    chunk_ref[0] = acc


def _extract_body(candx_ref, candy_ref, candz_ref, ic_ref, cent_ref,
                  idx_ref, *, group_size, row_block, n_points):
    b = pl.program_id(0)
    candx = candx_ref[0]
    candy = candy_ref[0]
    candz = candz_ref[0]
    ic = ic_ref[0]
    cx = cent_ref[0][:, 0:1]
    cy = cent_ref[0][:, 1:2]
    cz = cent_ref[0][:, 2:3]

    dx = candx - cx
    dy = candy - cy
    dz = candz - cz
    dc = dx * dx + dy * dy
    dc = dc + dz * dz

    iota_s = jax.lax.broadcasted_iota(jnp.int32, (row_block, group_size), 1)
    acc = jnp.zeros((row_block, group_size), dtype=jnp.int32)
    for s in range(group_size):
        mv = jnp.min(dc, axis=1, keepdims=True)
        j = jnp.min(jnp.where(dc == mv, ic, n_points), axis=1, keepdims=True)
        acc = acc + jnp.where(iota_s == s, j, 0)
        if s + 1 < group_size:
            dc = jnp.where(ic == j, _INF, dc)
    idx_ref[0] = acc + b * n_points


def _sub_body(rows_ref, cent_ref, out_ref):
    out_ref[0] = rows_ref[0] - cent_ref[0][:, None, :]


def _sc_gather(table, idx2d, row_w, buf_rows=128):
    info = plsc.get_sparse_core_info()
    nc, ns = info.num_cores, info.num_subcores
    nw = nc * ns
    total = idx2d.shape[0] * 128
    rows_per_w = total // nw
    chunks_per_w = rows_per_w // 128
    mesh = plsc.VectorSubcoreMesh(core_axis_name="c", subcore_axis_name="s")

    @functools.partial(
        pl.kernel, mesh=mesh,
        compiler_params=pltpu.CompilerParams(use_tc_tiling_on_sc=False),
        out_type=jax.ShapeDtypeStruct((total, row_w), jnp.float32),
        scratch_types=[
            pltpu.VMEM((chunks_per_w, 128), jnp.int32),
            pltpu.VMEM((2, buf_rows, row_w), jnp.float32),
            pltpu.SemaphoreType.DMA,
            pltpu.SemaphoreType.DMA,
        ],
    )
    def k(table_hbm, idx_hbm, out_hbm, idx_v, rows_v, sem0, sem1):
        wid = lax.axis_index("s") * nc + lax.axis_index("c")
        pltpu.sync_copy(idx_hbm.at[pl.ds(wid * chunks_per_w, chunks_per_w)],
                        idx_v)
        sems = [sem0, sem1]
        gathers = []
        outs = []
        for c in range(chunks_per_w):
            p = c % 2
            if c >= 2:
                outs[c - 2].wait()
            gathers.append(pltpu.async_copy(
                table_hbm.at[idx_v.at[c]], rows_v.at[p], sems[p]))
            gathers[c].wait()
            outs.append(pltpu.async_copy(
                rows_v.at[p],
                out_hbm.at[pl.ds(wid * rows_per_w + c * 128, 128)],
                sems[p]))
        outs[chunks_per_w - 2].wait()
        outs[chunks_per_w - 1].wait()

    return k(table, idx2d)


def _group(xyz, n_centers, group_size, interpret=False):
    B, N, _ = xyz.shape
    n_chunks = N // _CH
    xyzT = jnp.transpose(xyz, (2, 0, 1))

    cent3 = pl.pallas_call(
        functools.partial(_fps_body, n_centers=n_centers),
        out_shape=jax.ShapeDtypeStruct((3, B, n_centers), jnp.float32),
        interpret=interpret,
    )(xyzT)

    centb = jnp.transpose(cent3, (1, 0, 2))
    center = jnp.transpose(cent3, (1, 2, 0))

    chunks = pl.pallas_call(
        functools.partial(_chunksel_body, n_chunks=n_chunks),
        grid=(B,),
        in_specs=[
            pl.BlockSpec((1, N, 3), lambda b: (b, 0, 0)),
            pl.BlockSpec((1, 3, n_centers), lambda b: (b, 0, 0)),
        ],
        out_specs=pl.BlockSpec((1, _NSEL, n_centers), lambda b: (b, 0, 0)),
        out_shape=jax.ShapeDtypeStruct((B, _NSEL, n_centers), jnp.int32),
        compiler_params=pltpu.CompilerParams(
            dimension_semantics=("parallel",),
        ),
        interpret=interpret,
    )(xyz, centb)

    chunks_l = jnp.transpose(chunks, (0, 2, 1))

    ncand = _NSEL * _CH
    table3 = xyzT.reshape(3 * B * n_chunks, _CH)
    cg = chunks_l + (jnp.arange(B, dtype=jnp.int32) * n_chunks)[:, None,
                                                                None]
    idx3 = jnp.stack([cg, cg + B * n_chunks, cg + 2 * B * n_chunks])
    cand = _sc_gather(table3, idx3.reshape(-1, 128), _CH)
    cand4 = cand.reshape(3, B, n_centers, ncand)

    ic_arr = (chunks_l[..., None] * _CH
              + jnp.arange(_CH, dtype=jnp.int32)).reshape(
                  B, n_centers, ncand)

    RB = min(256, n_centers)
    idx = pl.pallas_call(
        functools.partial(_extract_body, group_size=group_size,
                          row_block=RB, n_points=N),
        grid=(B, n_centers // RB),
        in_specs=[
            pl.BlockSpec((1, RB, ncand), lambda b, r: (b, r, 0)),
            pl.BlockSpec((1, RB, ncand), lambda b, r: (b, r, 0)),
            pl.BlockSpec((1, RB, ncand), lambda b, r: (b, r, 0)),
            pl.BlockSpec((1, RB, ncand), lambda b, r: (b, r, 0)),
            pl.BlockSpec((1, RB, 3), lambda b, r: (b, r, 0)),
        ],
        out_specs=pl.BlockSpec((1, RB, group_size), lambda b, r: (b, r, 0)),
        out_shape=jax.ShapeDtypeStruct((B, n_centers, group_size),
                                       jnp.int32),
        compiler_params=pltpu.CompilerParams(
            dimension_semantics=("parallel", "parallel"),
        ),
        interpret=interpret,
    )(cand4[0], cand4[1], cand4[2], ic_arr, center)

    table = jnp.pad(xyz.reshape(B * N, 3), ((0, 0), (0, _PAD - 3)))
    rows = _sc_gather(table, idx.reshape(-1, 128), _PAD)
    rows4 = rows.reshape(B, n_centers, group_size, _PAD)

    cent_p = jnp.pad(center, ((0, 0), (0, 0), (0, _PAD - 3)))

    nb = pl.pallas_call(
        _sub_body,
        grid=(B,),
        in_specs=[
            pl.BlockSpec((1, n_centers, group_size, _PAD),
                         lambda b: (b, 0, 0, 0)),
            pl.BlockSpec((1, n_centers, _PAD), lambda b: (b, 0, 0)),
        ],
        out_specs=pl.BlockSpec((1, n_centers, group_size, _PAD),
                               lambda b: (b, 0, 0, 0)),
        out_shape=jax.ShapeDtypeStruct((B, n_centers, group_size, _PAD),
                                       jnp.float32),
        compiler_params=pltpu.CompilerParams(
            dimension_semantics=("parallel",),
        ),
        interpret=interpret,
    )(rows4, cent_p)

    return (nb[..., :3], center)


def kernel(xyz):
    return _group(xyz, 256, 32)


# speedup vs baseline: 1.1269x; 1.0008x over previous
import functools

import jax
import jax.numpy as jnp
from jax import lax
from jax.experimental import pallas as pl
from jax.experimental.pallas import tpu as pltpu
from jax.experimental.pallas import tpu_sc as plsc

_BIG = 1e10
_INF = float("inf")
_PAD = 16
_CH = 32
_NSEL = 32


def _fps_body(xyz_ref, cent_ref, *, n_centers):
    x = xyz_ref[0]
    y = xyz_ref[1]
    z = xyz_ref[2]
    B, N = x.shape
    iota = jax.lax.broadcasted_iota(jnp.int32, (B, N), 1)
    iota_g = jax.lax.broadcasted_iota(jnp.int32, (B, n_centers), 1)

    def body(i, state):
        dists, lx, ly, lz, ax, ay, az = state
        dx = x - lx
        dy = y - ly
        dz = z - lz
        d = dx * dx + dy * dy
        d = d + dz * dz
        dists = jnp.minimum(dists, d)
        j = jnp.argmax(dists, axis=1).astype(jnp.int32)[:, None]
        onehot = iota == j
        lx = jnp.sum(jnp.where(onehot, x, 0.0), axis=1, keepdims=True)
        ly = jnp.sum(jnp.where(onehot, y, 0.0), axis=1, keepdims=True)
        lz = jnp.sum(jnp.where(onehot, z, 0.0), axis=1, keepdims=True)
        sel = iota_g == i
        ax = ax + jnp.where(sel, lx, 0.0)
        ay = ay + jnp.where(sel, ly, 0.0)
        az = az + jnp.where(sel, lz, 0.0)
        return (dists, lx, ly, lz, ax, ay, az)

    dists0 = jnp.full((B, N), _BIG, dtype=jnp.float32)
    sel0 = iota_g == 0
    ax0 = jnp.where(sel0, x[:, 0:1], 0.0)
    ay0 = jnp.where(sel0, y[:, 0:1], 0.0)
    az0 = jnp.where(sel0, z[:, 0:1], 0.0)
    out = jax.lax.fori_loop(
        1, n_centers, body,
        (dists0, x[:, 0:1], y[:, 0:1], z[:, 0:1], ax0, ay0, az0))
    cent_ref[0] = out[4]
    cent_ref[1] = out[5]
    cent_ref[2] = out[6]


def _chunksel_body(xyz_ref, cent_ref, chunk_ref, *, n_chunks):
    x = xyz_ref[0][:, 0:1]
    y = xyz_ref[0][:, 1:2]
    z = xyz_ref[0][:, 2:3]
    cx = cent_ref[0, 0:1, :]
    cy = cent_ref[0, 1:2, :]
    cz = cent_ref[0, 2:3, :]
    N = x.shape[0]
    G = cx.shape[1]
    b = pl.program_id(0)

    dx = x - cx
    dy = y - cy
    dz = z - cz
    d = dx * dx + dy * dy
    d = d + dz * dz

    m = jnp.min(d.reshape(n_chunks, _CH, G), axis=1)
    iota_c = jax.lax.broadcasted_iota(jnp.int32, (n_chunks, G), 0)
    iota_slot = jax.lax.broadcasted_iota(jnp.int32, (_NSEL, G), 0)

    acc = jnp.zeros((_NSEL, G), dtype=jnp.int32)
    for s in range(_NSEL):
        mv = jnp.min(m, axis=0, keepdims=True)
        j = jnp.min(jnp.where(m == mv, iota_c, n_chunks), axis=0,
                    keepdims=True)
        acc = acc + jnp.where(iota_slot == s, j, 0)
        if s + 1 < _NSEL:
            m = jnp.where(iota_c == j, _INF, m)
    chunk_ref[0] = acc


def _extract_body(candx_ref, candy_ref, candz_ref, ic_ref, cent_ref,
                  idx_ref, *, group_size, row_block, n_points):
    b = pl.program_id(0)
    candx = candx_ref[0]
    candy = candy_ref[0]
    candz = candz_ref[0]
    ic = ic_ref[0]
    cx = cent_ref[0][:, 0:1]
    cy = cent_ref[0][:, 1:2]
    cz = cent_ref[0][:, 2:3]

    dx = candx - cx
    dy = candy - cy
    dz = candz - cz
    dc = dx * dx + dy * dy
    dc = dc + dz * dz

    iota_s = jax.lax.broadcasted_iota(jnp.int32, (row_block, group_size), 1)
    acc = jnp.zeros((row_block, group_size), dtype=jnp.int32)
    for s in range(group_size):
        mv = jnp.min(dc, axis=1, keepdims=True)
        j = jnp.min(jnp.where(dc == mv, ic, n_points), axis=1, keepdims=True)
        acc = acc + jnp.where(iota_s == s, j, 0)
        if s + 1 < group_size:
            dc = jnp.where(ic == j, _INF, dc)
    idx_ref[0] = acc + b * n_points


def _sub_body(rows_ref, cent_ref, out_ref):
    out_ref[0] = rows_ref[0] - cent_ref[0][:, None, :]


def _sc_gather(table, idx2d, row_w, buf_rows=128):
    info = plsc.get_sparse_core_info()
    nc, ns = info.num_cores, info.num_subcores
    nw = nc * ns
    total = idx2d.shape[0] * 128
    rows_per_w = total // nw
    chunks_per_w = rows_per_w // 128
    mesh = plsc.VectorSubcoreMesh(core_axis_name="c", subcore_axis_name="s")

    @functools.partial(
        pl.kernel, mesh=mesh,
        compiler_params=pltpu.CompilerParams(use_tc_tiling_on_sc=False),
        out_type=jax.ShapeDtypeStruct((total, row_w), jnp.float32),
        scratch_types=[
            pltpu.VMEM((chunks_per_w, 128), jnp.int32),
            pltpu.VMEM((2, buf_rows, row_w), jnp.float32),
            pltpu.SemaphoreType.DMA,
            pltpu.SemaphoreType.DMA,
        ],
    )
    def k(table_hbm, idx_hbm, out_hbm, idx_v, rows_v, sem0, sem1):
        wid = lax.axis_index("s") * nc + lax.axis_index("c")
        pltpu.sync_copy(idx_hbm.at[pl.ds(wid * chunks_per_w, chunks_per_w)],
                        idx_v)
        sems = [sem0, sem1]
        gathers = []
        outs = []
        for c in range(chunks_per_w):
            p = c % 2
            if c >= 2:
                outs[c - 2].wait()
            gathers.append(pltpu.async_copy(
                table_hbm.at[idx_v.at[c]], rows_v.at[p], sems[p]))
            gathers[c].wait()
            outs.append(pltpu.async_copy(
                rows_v.at[p],
                out_hbm.at[pl.ds(wid * rows_per_w + c * 128, 128)],
                sems[p]))
        outs[chunks_per_w - 2].wait()
        outs[chunks_per_w - 1].wait()

    return k(table, idx2d)


def _group(xyz, n_centers, group_size):
    B, N, _ = xyz.shape
    n_chunks = N // _CH
    xyzT = jnp.transpose(xyz, (2, 0, 1))

    cent3 = pl.pallas_call(
        functools.partial(_fps_body, n_centers=n_centers),
        out_shape=jax.ShapeDtypeStruct((3, B, n_centers), jnp.float32),
    )(xyzT)

    centb = jnp.transpose(cent3, (1, 0, 2))
    center = jnp.transpose(cent3, (1, 2, 0))

    chunks = pl.pallas_call(
        functools.partial(_chunksel_body, n_chunks=n_chunks),
        grid=(B,),
        in_specs=[
            pl.BlockSpec((1, N, 3), lambda b: (b, 0, 0)),
            pl.BlockSpec((1, 3, n_centers), lambda b: (b, 0, 0)),
        ],
        out_specs=pl.BlockSpec((1, _NSEL, n_centers), lambda b: (b, 0, 0)),
        out_shape=jax.ShapeDtypeStruct((B, _NSEL, n_centers), jnp.int32),
        compiler_params=pltpu.CompilerParams(
            dimension_semantics=("parallel",),
        ),
    )(xyz, centb)

    chunks_l = jnp.transpose(chunks, (0, 2, 1))

    ncand = _NSEL * _CH
    table3 = xyzT.reshape(3 * B * n_chunks, _CH)
    cg = chunks_l + (jnp.arange(B, dtype=jnp.int32) * n_chunks)[:, None,
                                                                None]
    idx3 = jnp.stack([cg, cg + B * n_chunks, cg + 2 * B * n_chunks])
    cand = _sc_gather(table3, idx3.reshape(-1, 128), _CH)
    cand4 = cand.reshape(3, B, n_centers, ncand)

    ic_arr = (chunks_l[..., None] * _CH
              + jnp.arange(_CH, dtype=jnp.int32)).reshape(
                  B, n_centers, ncand)

    RB = min(256, n_centers)
    idx = pl.pallas_call(
        functools.partial(_extract_body, group_size=group_size,
                          row_block=RB, n_points=N),
        grid=(B, n_centers // RB),
        in_specs=[
            pl.BlockSpec((1, RB, ncand), lambda b, r: (b, r, 0)),
            pl.BlockSpec((1, RB, ncand), lambda b, r: (b, r, 0)),
            pl.BlockSpec((1, RB, ncand), lambda b, r: (b, r, 0)),
            pl.BlockSpec((1, RB, ncand), lambda b, r: (b, r, 0)),
            pl.BlockSpec((1, RB, 3), lambda b, r: (b, r, 0)),
        ],
        out_specs=pl.BlockSpec((1, RB, group_size), lambda b, r: (b, r, 0)),
        out_shape=jax.ShapeDtypeStruct((B, n_centers, group_size),
                                       jnp.int32),
        compiler_params=pltpu.CompilerParams(
            dimension_semantics=("parallel", "parallel"),
        ),
    )(cand4[0], cand4[1], cand4[2], ic_arr, center)

    table = jnp.pad(xyz.reshape(B * N, 3), ((0, 0), (0, _PAD - 3)))
    rows = _sc_gather(table, idx.reshape(-1, 128), _PAD)
    rows4 = rows.reshape(B, n_centers, group_size, _PAD)

    cent_p = jnp.pad(center, ((0, 0), (0, 0), (0, _PAD - 3)))

    nb = pl.pallas_call(
        _sub_body,
        grid=(B,),
        in_specs=[
            pl.BlockSpec((1, n_centers, group_size, _PAD),
                         lambda b: (b, 0, 0, 0)),
            pl.BlockSpec((1, n_centers, _PAD), lambda b: (b, 0, 0)),
        ],
        out_specs=pl.BlockSpec((1, n_centers, group_size, _PAD),
                               lambda b: (b, 0, 0, 0)),
        out_shape=jax.ShapeDtypeStruct((B, n_centers, group_size, _PAD),
                                       jnp.float32),
        compiler_params=pltpu.CompilerParams(
            dimension_semantics=("parallel",),
        ),
    )(rows4, cent_p)

    return (nb[..., :3], center)


def kernel(xyz):
    return _group(xyz, 256, 32)
